# baseline (speedup 1.0000x reference)
import functools

import jax
import jax.numpy as jnp
from jax import lax
from jax.experimental import pallas as pl
from jax.experimental.pallas import tpu as pltpu

N_HEADS = 8
D_K = 64
D_MODEL = 300
D_HID = 300


def _layer_norm(x, g, b):
    m = jnp.mean(x, -1, keepdims=True)
    v = jnp.var(x, -1, keepdims=True)
    return (x - m) / jnp.sqrt(v + 1e-5) * g + b


def _gcn_layer(x, src, dst, W, b, n):
    h = x @ W
    loop = jnp.arange(n, dtype=src.dtype)
    s = jnp.concatenate([src, loop])
    d = jnp.concatenate([dst, loop])
    deg = jax.ops.segment_sum(jnp.ones(d.shape[0], h.dtype), d, num_segments=n)
    dis = jnp.where(deg > 0, 1.0 / jnp.sqrt(jnp.maximum(deg, 1e-12)), 0.0)
    norm = dis[s] * dis[d]
    return jax.ops.segment_sum(h[s] * norm[:, None], d, num_segments=n) + b


def _attention_block(Q, K, V, p):
    b, l, _ = Q.shape
    def heads(X, W):
        return (X @ W).reshape(b, l, N_HEADS, D_K).transpose(0, 2, 1, 3)
    q = heads(Q, p['Wq'])
    k = heads(K, p['Wk'])
    v = heads(V, p['Wv'])
    sc = jnp.einsum('bhid,bhjd->bhij', q, k) / jnp.sqrt(jnp.float32(D_K))
    def proj(att):
        o = jnp.einsum('bhij,bhjd->bhid', att, v)
        o = o.transpose(0, 2, 1, 3).reshape(b, l, N_HEADS * D_K) @ p['Wo']
        return _layer_norm(Q + o, p['ln_g'], p['ln_b'])
    return proj(jax.nn.softmax(sc, -1)), proj(jax.nn.softmax(-sc, -1))


def _gru_dir(x, p, reverse):
    xs = jnp.swapaxes(x, 0, 1)
    if reverse:
        xs = xs[::-1]
    def step(h, xt):
        gx = xt @ p['Wx'] + p['bx']
        gh = h @ p['Wh'] + p['bh']
        xr, xz, xn = jnp.split(gx, 3, -1)
        hr, hz, hn = jnp.split(gh, 3, -1)
        r = jax.nn.sigmoid(xr + hr)
        z = jax.nn.sigmoid(xz + hz)
        nn_ = jnp.tanh(xn + r * hn)
        h2 = (1.0 - z) * nn_ + z * h
        return h2, h2
    h0 = jnp.zeros((x.shape[0], D_HID), x.dtype)
    _, ys = jax.lax.scan(step, h0, xs)
    if reverse:
        ys = ys[::-1]
    return jnp.swapaxes(ys, 0, 1)


def _fc_head_body(feats_ref, w1_ref, b1_ref, w2_ref, b2_ref, out_ref):
    h = jnp.maximum(feats_ref[...] @ w1_ref[...] + b1_ref[...], 0.0)
    out_ref[...] = h @ w2_ref[...] + b2_ref[...]


def _fc_head(feats, w1, b1, w2, b2):
    B = feats.shape[0]
    b1r = b1.reshape(1, -1)
    b2r = b2.reshape(1, -1)
    return pl.pallas_call(
        _fc_head_body,
        out_shape=jax.ShapeDtypeStruct((B, w2.shape[1]), jnp.float32),
    )(feats, w1, b1r, w2, b2r)


def kernel(x, params, edge_index, root_index, text):
    n = x.shape[0]
    src, dst = edge_index[0], edge_index[1]
    g = _gcn_layer(x, src, dst, params['gcn1_W'], params['gcn1_b'], n)
    g = _gcn_layer(g, src, dst, params['gcn2_W'], params['gcn2_b'], n)
    g = g[root_index]
    t = params['emb'][text[root_index]]
    p1, n1 = _attention_block(t, t, t, params['att1'])
    t = 0.5 * (p1 + n1)
    p2, n2 = _attention_block(t, t, t, params['att2'])
    t = 0.5 * (p2 + n2)
    fwd = _gru_dir(t, params['gru_f'], False)
    bwd = _gru_dir(t, params['gru_b'], True)
    seq = jnp.mean(jnp.concatenate([fwd, bwd], -1), axis=1)
    feats = jnp.concatenate([seq, g], -1)
    return _fc_head(feats, params['fc1_W'], params['fc1_b'],
                    params['fc2_W'], params['fc2_b'])


# speedup vs baseline: 3.2697x; 3.2697x over previous
import functools

import jax
import jax.numpy as jnp
from jax import lax
from jax.experimental import pallas as pl
from jax.experimental.pallas import tpu as pltpu
from jax.experimental.pallas import tpu_sc as plsc

N_HEADS = 8
D_K = 64
D_MODEL = 300
D_HID = 300

NN = 10000
NE = 160000
NEP = 163840
DH = 160
DP = 320
ACC_ROWS = 10112
DUMMY = NN

_mesh = functools.partial(
    plsc.VectorSubcoreMesh, core_axis_name="c", subcore_axis_name="s")

_SC_PARAMS = pltpu.CompilerParams(use_tc_tiling_on_sc=False)


def _deg_body(dst_hbm, out_hbm, dstb, ones_v, zbuf, acc_sh, _sem):
    c = lax.axis_index("c")
    s = lax.axis_index("s")

    def fill(i, _):
        ones_v[i, :] = jnp.full((16,), 1.0, jnp.float32)
        zbuf[i, :] = jnp.zeros((16,), jnp.float32)
        return _
    lax.fori_loop(0, 128, fill, 0)

    zb = s * 632
    for k in range(4):
        pltpu.sync_copy(zbuf.at[:, :], acc_sh.at[pl.ds(zb + 128 * k, 128), :])
    pltpu.sync_copy(zbuf.at[pl.ds(0, 120), :],
                    acc_sh.at[pl.ds(zb + 512, 120), :])
    plsc.subcore_barrier()

    ebase = (c * 16 + s) * (NEP // 32)
    def chunk(j, _):
        pltpu.sync_copy(dst_hbm.at[pl.ds(ebase + 128 * j, 128)], dstb.at[0])
        pltpu.sync_copy(ones_v, acc_sh.at[dstb.at[0]], add=True)
        return _
    lax.fori_loop(0, NEP // 32 // 128, chunk, 0)
    plsc.subcore_barrier()

    rb = s * 632
    ob = c * NN + s * 632
    for k in range(4):
        pltpu.sync_copy(acc_sh.at[pl.ds(rb + 128 * k, 128), :],
                        out_hbm.at[pl.ds(ob + 128 * k, 128), :])
    @pl.when(s < 15)
    def _():
        pltpu.sync_copy(acc_sh.at[pl.ds(rb + 512, 120), :],
                        out_hbm.at[pl.ds(ob + 512, 120), :])
    @pl.when(s == 15)
    def _():
        pltpu.sync_copy(acc_sh.at[pl.ds(rb + 512, 8), :],
                        out_hbm.at[pl.ds(ob + 512, 8), :])


def _deg_parts(dstp):
    return pl.kernel(
        _deg_body,
        out_type=jax.ShapeDtypeStruct((2 * NN, 16), jnp.float32),
        mesh=_mesh(),
        scratch_types=[
            pltpu.VMEM((2, 128), jnp.int32),
            pltpu.VMEM((128, 16), jnp.float32),
            pltpu.VMEM((128, 16), jnp.float32),
            pltpu.VMEM_SHARED((ACC_ROWS, 16), jnp.float32),
            pltpu.SemaphoreType.DMA,
        ],
        compiler_params=_SC_PARAMS,
    )(dstp)


def _agg_body(srcs_hbm, dst_hbm, hp_hbm, out_hbm,
              srcb, dstb, rows_v, acc_sh, gsem):
    c = lax.axis_index("c")
    s = lax.axis_index("s")

    def fill(i, _):
        for j in range(DH // 16):
            rows_v[i, pl.ds(16 * j, 16)] = jnp.zeros((16,), jnp.float32)
        return _
    lax.fori_loop(0, 128, fill, 0)

    zb = s * 632
    for k in range(4):
        pltpu.sync_copy(rows_v.at[:, :], acc_sh.at[pl.ds(zb + 128 * k, 128), :])
    pltpu.sync_copy(rows_v.at[pl.ds(0, 120), :],
                    acc_sh.at[pl.ds(zb + 512, 120), :])
    plsc.subcore_barrier()

    per_tile = NEP // 16
    def chunk(j, _):
        eoff = c * NEP + s * per_tile + 128 * j
        doff = s * per_tile + 128 * j
        pltpu.sync_copy(srcs_hbm.at[pl.ds(eoff, 128)], srcb.at[0])
        pltpu.sync_copy(dst_hbm.at[pl.ds(doff, 128)], dstb.at[0])
        pltpu.async_copy(hp_hbm.at[srcb.at[0]], rows_v, gsem).wait()
        pltpu.sync_copy(rows_v, acc_sh.at[dstb.at[0]], add=True)
        return _
    lax.fori_loop(0, per_tile // 128, chunk, 0)
    plsc.subcore_barrier()

    rb = s * 632
    ob = c * NN + s * 632
    for k in range(4):
        pltpu.sync_copy(acc_sh.at[pl.ds(rb + 128 * k, 128), :],
                        out_hbm.at[pl.ds(ob + 128 * k, 128), :])
    @pl.when(s < 15)
    def _():
        pltpu.sync_copy(acc_sh.at[pl.ds(rb + 512, 120), :],
                        out_hbm.at[pl.ds(ob + 512, 120), :])
    @pl.when(s == 15)
    def _():
        pltpu.sync_copy(acc_sh.at[pl.ds(rb + 512, 8), :],
                        out_hbm.at[pl.ds(ob + 512, 8), :])


def _agg(srcs2, dstp, hp_stacked):
    return pl.kernel(
        _agg_body,
        out_type=jax.ShapeDtypeStruct((2 * NN, DH), jnp.float32),
        mesh=_mesh(),
        scratch_types=[
            pltpu.VMEM((2, 128), jnp.int32),
            pltpu.VMEM((2, 128), jnp.int32),
            pltpu.VMEM((128, DH), jnp.float32),
            pltpu.VMEM_SHARED((ACC_ROWS, DH), jnp.float32),
            pltpu.SemaphoreType.DMA,
        ],
        compiler_params=_SC_PARAMS,
    )(srcs2, dstp, hp_stacked)


def _layer_norm(x, g, b):
    m = jnp.mean(x, -1, keepdims=True)
    v = jnp.var(x, -1, keepdims=True)
    return (x - m) / jnp.sqrt(v + 1e-5) * g + b


def _attention_block(Q, K, V, p):
    b, l, _ = Q.shape
    def heads(X, W):
        return (X @ W).reshape(b, l, N_HEADS, D_K).transpose(0, 2, 1, 3)
    q = heads(Q, p['Wq'])
    k = heads(K, p['Wk'])
    v = heads(V, p['Wv'])
    sc = jnp.einsum('bhid,bhjd->bhij', q, k) / jnp.sqrt(jnp.float32(D_K))
    def proj(att):
        o = jnp.einsum('bhij,bhjd->bhid', att, v)
        o = o.transpose(0, 2, 1, 3).reshape(b, l, N_HEADS * D_K) @ p['Wo']
        return _layer_norm(Q + o, p['ln_g'], p['ln_b'])
    return proj(jax.nn.softmax(sc, -1)), proj(jax.nn.softmax(-sc, -1))


def _gru_dir(x, p, reverse):
    xs = jnp.swapaxes(x, 0, 1)
    if reverse:
        xs = xs[::-1]
    def step(h, xt):
        gx = xt @ p['Wx'] + p['bx']
        gh = h @ p['Wh'] + p['bh']
        xr, xz, xn = jnp.split(gx, 3, -1)
        hr, hz, hn = jnp.split(gh, 3, -1)
        r = jax.nn.sigmoid(xr + hr)
        z = jax.nn.sigmoid(xz + hz)
        nn_ = jnp.tanh(xn + r * hn)
        h2 = (1.0 - z) * nn_ + z * h
        return h2, h2
    h0 = jnp.zeros((x.shape[0], D_HID), x.dtype)
    _, ys = jax.lax.scan(step, h0, xs)
    if reverse:
        ys = ys[::-1]
    return jnp.swapaxes(ys, 0, 1)


def _fc_head_body(feats_ref, w1_ref, b1_ref, w2_ref, b2_ref, out_ref):
    h = jnp.maximum(feats_ref[...] @ w1_ref[...] + b1_ref[...], 0.0)
    out_ref[...] = h @ w2_ref[...] + b2_ref[...]


def _fc_head(feats, w1, b1, w2, b2):
    B = feats.shape[0]
    return pl.pallas_call(
        _fc_head_body,
        out_shape=jax.ShapeDtypeStruct((B, w2.shape[1]), jnp.float32),
    )(feats, w1, b1.reshape(1, -1), w2, b2.reshape(1, -1))


def _pad_w(W, rows, cols):
    return jnp.pad(W, ((0, rows - W.shape[0]), (0, cols - W.shape[1])))


def kernel(x, params, edge_index, root_index, text):
    src, dst = edge_index[0], edge_index[1]
    fill = jnp.zeros((NEP - NE,), jnp.int32)
    srcp = jnp.concatenate([src, fill])
    dstp = jnp.concatenate([dst, jnp.full((NEP - NE,), DUMMY, jnp.int32)])
    srcs2 = jnp.concatenate([srcp, srcp + NN])

    parts = _deg_parts(dstp)
    deg = parts[:NN, 0] + parts[NN:, 0] + 1.0
    dis = lax.rsqrt(deg)[:, None]

    W1 = _pad_w(params['gcn1_W'], D_MODEL, DP)
    b1 = jnp.pad(params['gcn1_b'], (0, DP - 300))
    W2 = _pad_w(params['gcn2_W'], DP, DP)
    b2 = jnp.pad(params['gcn2_b'], (0, DP - 300))

    hp1 = (x @ W1) * dis
    hp1s = jnp.concatenate([hp1[:, :DH], hp1[:, DH:]], axis=0)
    acc1 = _agg(srcs2, dstp, hp1s)
    acc1f = jnp.concatenate([acc1[:NN], acc1[NN:]], axis=1)
    g1 = dis * (acc1f + hp1) + b1

    hp2 = (g1 @ W2) * dis
    hp2s = jnp.concatenate([hp2[:, :DH], hp2[:, DH:]], axis=0)
    acc2 = _agg(srcs2, dstp, hp2s)
    acc2f = jnp.concatenate([acc2[:NN], acc2[NN:]], axis=1)
    g2 = dis * (acc2f + hp2) + b2

    g = g2[root_index][:, :300]

    t = params['emb'][text[root_index]]
    p1, n1 = _attention_block(t, t, t, params['att1'])
    t = 0.5 * (p1 + n1)
    p2, n2 = _attention_block(t, t, t, params['att2'])
    t = 0.5 * (p2 + n2)
    fwd = _gru_dir(t, params['gru_f'], False)
    bwd = _gru_dir(t, params['gru_b'], True)
    seq = jnp.mean(jnp.concatenate([fwd, bwd], -1), axis=1)
    feats = jnp.concatenate([seq, g], -1)
    return _fc_head(feats, params['fc1_W'], params['fc1_b'],
                    params['fc2_W'], params['fc2_b'])
